# Initial kernel scaffold; baseline (speedup 1.0000x reference)
import jax
import jax.numpy as jnp
from jax.experimental import pallas as pl


def kernel(x, edge_index, batch, W_gnn, b_gnn, W_imp, b_imp, W1, b1, W2, b2):
    raise NotImplementedError("write your pallas kernel here")


# speedup vs baseline: 6.4880x; 6.4880x over previous
import functools

import jax
import jax.numpy as jnp
from jax import lax
from jax.experimental import pallas as pl
from jax.experimental.pallas import tpu as pltpu
from jax.experimental.pallas import tpu_sc as plsc

N = 10000
E = 320000
D = 128
G = 128

NC = 2
NS = 16
NW = NC * NS
EPT = E // NW
CH = 80
NCH = EPT // CH
NPAD = 10240
NPS = NPAD // NS

R = 400
NB = N // R

def _sc_body(x_hbm, src_hbm, dst_hbm, zeros_hbm, out_hbm,
             agg_sh, src_v, dst_v, rows_v, sem):
    cid = lax.axis_index("c")
    sid = lax.axis_index("s")
    w = cid * NS + sid
    r0 = sid * NPS
    pltpu.sync_copy(zeros_hbm.at[pl.ds(r0, NPS)], agg_sh.at[pl.ds(r0, NPS)])
    pltpu.sync_copy(src_hbm.at[w], src_v)
    pltpu.sync_copy(dst_hbm.at[w], dst_v)
    plsc.subcore_barrier()

    def chunk(ci, carry):
        pltpu.async_copy(x_hbm.at[src_v.at[ci]], rows_v, sem).wait()
        pltpu.sync_copy(rows_v, agg_sh.at[dst_v.at[ci]], add=True)
        return carry

    lax.fori_loop(0, NCH, chunk, 0)
    plsc.subcore_barrier()
    pltpu.sync_copy(agg_sh.at[pl.ds(r0, NPS)], out_hbm.at[cid, pl.ds(r0, NPS)])


@functools.cache
def _sc_scatter_add():
    mesh = plsc.VectorSubcoreMesh(
        core_axis_name="c", subcore_axis_name="s",
        num_cores=NC, num_subcores=NS)
    return pl.kernel(
        _sc_body,
        out_type=jax.ShapeDtypeStruct((NC, NPAD, D), jnp.float32),
        mesh=mesh,
        scratch_types=[
            pltpu.VMEM_SHARED((NPAD, D), jnp.float32),
            pltpu.VMEM((NCH, CH), jnp.int32),
            pltpu.VMEM((NCH, CH), jnp.int32),
            pltpu.VMEM((CH, D), jnp.float32),
            pltpu.SemaphoreType.DMA,
        ],
    )


def _tc1_body(p0_ref, p1_ref, batch_ref, wgnn_ref, bgnn_ref, wimp_ref,
              bimp_ref, h_ref, ni_ref, segmax_ref):
    i = pl.program_id(0)
    agg = p0_ref[...] + p1_ref[...]
    h = jnp.maximum(
        lax.dot_general(agg, wgnn_ref[...], (((1,), (0,)), ((), ())),
                        preferred_element_type=jnp.float32) + bgnn_ref[...],
        0.0)
    h_ref[...] = h
    s = jnp.sum(agg * wimp_ref[...], axis=1, keepdims=True) + bimp_ref[...]
    ni = jax.nn.sigmoid(s)
    ni_ref[...] = ni
    lanes = lax.broadcasted_iota(jnp.int32, (R, G), 1)
    m = batch_ref[...] == lanes
    vals = jnp.where(m, ni, -jnp.inf)
    blockmax = jnp.max(vals, axis=0, keepdims=True)

    @pl.when(i == 0)
    def _init():
        segmax_ref[...] = jnp.full((8, G), -jnp.inf, jnp.float32)

    segmax_ref[...] = jnp.maximum(segmax_ref[...],
                                  jnp.broadcast_to(blockmax, (8, G)))


def _tc2_body(h_ref, ni_ref, batch_ref, segmax_ref, w1_ref, b1_ref, w2_ref,
              b2_ref, xw_ref, xg_ref, sums_ref, counts_ref):
    i = pl.program_id(0)
    lanes = lax.broadcasted_iota(jnp.int32, (R, G), 1)
    m = batch_ref[...] == lanes
    mf = m.astype(jnp.float32)
    segb = jnp.broadcast_to(segmax_ref[0:1, :], (R, G))
    out = jnp.sum(jnp.where(m, segb, 0.0), axis=1, keepdims=True)
    ni = ni_ref[:, 0:1]
    imp = ni / (out * 10.0) + 0.9
    xw = h_ref[...] * imp
    xw_ref[...] = xw

    @pl.when(i == 0)
    def _init():
        sums_ref[...] = jnp.zeros((G, D), jnp.float32)
        counts_ref[...] = jnp.zeros((G, D), jnp.float32)

    sums_ref[...] += lax.dot_general(mf, xw, (((0,), (0,)), ((), ())),
                                     preferred_element_type=jnp.float32)
    counts_ref[...] += lax.dot_general(mf, jnp.ones((R, D), jnp.float32),
                                       (((0,), (0,)), ((), ())),
                                       preferred_element_type=jnp.float32)

    @pl.when(i == NB - 1)
    def _final():
        xg = sums_ref[...] / jnp.maximum(counts_ref[...], 1.0)
        xg1 = jnp.maximum(
            lax.dot_general(xg, w1_ref[...], (((1,), (0,)), ((), ())),
                            preferred_element_type=jnp.float32) + b1_ref[...],
            0.0)
        xg_ref[...] = lax.dot_general(
            xg1, w2_ref[...], (((1,), (0,)), ((), ())),
            preferred_element_type=jnp.float32) + b2_ref[...]


_tc1 = pl.pallas_call(
    _tc1_body,
    grid=(NB,),
    in_specs=[
        pl.BlockSpec((R, D), lambda i: (i, 0)),
        pl.BlockSpec((R, D), lambda i: (i, 0)),
        pl.BlockSpec((R, G), lambda i: (i, 0)),
        pl.BlockSpec((D, D), lambda i: (0, 0)),
        pl.BlockSpec((1, D), lambda i: (0, 0)),
        pl.BlockSpec((1, D), lambda i: (0, 0)),
        pl.BlockSpec((1, D), lambda i: (0, 0)),
    ],
    out_specs=[
        pl.BlockSpec((R, D), lambda i: (i, 0)),
        pl.BlockSpec((R, G), lambda i: (i, 0)),
        pl.BlockSpec((8, G), lambda i: (0, 0)),
    ],
    out_shape=[
        jax.ShapeDtypeStruct((N, D), jnp.float32),
        jax.ShapeDtypeStruct((N, G), jnp.float32),
        jax.ShapeDtypeStruct((8, G), jnp.float32),
    ],
)

_tc2 = pl.pallas_call(
    _tc2_body,
    grid=(NB,),
    in_specs=[
        pl.BlockSpec((R, D), lambda i: (i, 0)),
        pl.BlockSpec((R, G), lambda i: (i, 0)),
        pl.BlockSpec((R, G), lambda i: (i, 0)),
        pl.BlockSpec((8, G), lambda i: (0, 0)),
        pl.BlockSpec((D, D), lambda i: (0, 0)),
        pl.BlockSpec((1, D), lambda i: (0, 0)),
        pl.BlockSpec((D, D), lambda i: (0, 0)),
        pl.BlockSpec((1, D), lambda i: (0, 0)),
    ],
    out_specs=[
        pl.BlockSpec((R, D), lambda i: (i, 0)),
        pl.BlockSpec((G, D), lambda i: (0, 0)),
    ],
    out_shape=[
        jax.ShapeDtypeStruct((N, D), jnp.float32),
        jax.ShapeDtypeStruct((G, D), jnp.float32),
    ],
    scratch_shapes=[
        pltpu.VMEM((G, D), jnp.float32),
        pltpu.VMEM((G, D), jnp.float32),
    ],
)


def kernel(x, edge_index, batch, W_gnn, b_gnn, W_imp, b_imp, W1, b1, W2, b2):
    src = edge_index[0].reshape(NW, NCH, CH)
    dst = edge_index[1].reshape(NW, NCH, CH)
    zeros = jnp.zeros((NPAD, D), jnp.float32)
    parts = _sc_scatter_add()(x, src, dst, zeros)
    batch_b = jnp.broadcast_to(batch[:, None], (N, G)).astype(jnp.int32)
    bgnn = jnp.broadcast_to(b_gnn[None, :], (1, D))
    wimp = jnp.broadcast_to(W_imp[:, 0][None, :], (1, D))
    bimp = jnp.broadcast_to(b_imp[None, :], (1, D))
    b1b = jnp.broadcast_to(b1[None, :], (1, D))
    b2b = jnp.broadcast_to(b2[None, :], (1, D))
    h, ni, segmax = _tc1(parts[0, :N], parts[1, :N], batch_b, W_gnn, bgnn, wimp, bimp)
    xw, x_graph = _tc2(h, ni, batch_b, segmax, W1, b1b, W2, b2b)
    return (x_graph, xw)
